# baseline (speedup 1.0000x reference)
import functools

import jax
import jax.numpy as jnp
from jax import lax
from jax.experimental import pallas as pl
from jax.experimental.pallas import tpu as pltpu
from jax.experimental.pallas import tpu_sc as plsc

ENT = 1000000
REL = 1000
H = 64
B = 4096
MARGIN = 1.0

NC = 2
NS = 16
NW = NC * NS
ROWS_PER_W = 4 * B // NW


CH = 32
NCHUNK = ROWS_PER_W // CH


def _sc_gather_body(tidx_hbm, sub_hbm, table_hbm, out_hbm,
                    tidx_v, sub_v, tb0, tb1, ob0, ob1,
                    sg0, sg1, so0, so1):
    wid = lax.axis_index("s") * NC + lax.axis_index("c")
    base = wid * ROWS_PER_W
    pltpu.sync_copy(tidx_hbm.at[pl.ds(base, ROWS_PER_W)], tidx_v)
    pltpu.sync_copy(sub_hbm.at[pl.ds(base, ROWS_PER_W)], sub_v)

    tb = [tb0, tb1]
    ob = [ob0, ob1]
    sg = [sg0, sg1]
    so = [so0, so1]

    def start_gather(c):
        return pltpu.async_copy(
            table_hbm.at[tidx_v.at[pl.ds(c * CH, CH)]], tb[c % 2], sg[c % 2])

    def extract(c):
        coff = c * CH
        for half in range(CH // 16):
            jv = lax.iota(jnp.int32, 16) + (half * 16)
            sv = sub_v[pl.ds(coff + half * 16, 16)]

            def col_body(k, _):
                kv = jnp.zeros((16,), jnp.int32) + k
                v = plsc.load_gather(tb[c % 2], [jv, sv, kv])
                plsc.store_scatter(ob[c % 2], [jv, kv], v)
                return 0

            lax.fori_loop(0, H, col_body, 0)

    g = [start_gather(0)]
    w = [None, None]
    for c in range(NCHUNK):
        g[c].wait()
        if c + 1 < NCHUNK:
            g.append(start_gather(c + 1))
        if c >= 2:
            w[c % 2].wait()
        extract(c)
        w[c % 2] = pltpu.async_copy(
            ob[c % 2], out_hbm.at[pl.ds(base + c * CH, CH)], so[c % 2])
    w[0].wait()
    w[1].wait()


def _sc_gather(idx, table):
    mesh = plsc.VectorSubcoreMesh(core_axis_name="c", subcore_axis_name="s")
    table3 = table.reshape(ENT // 8, 8, H)
    return pl.kernel(
        _sc_gather_body,
        mesh=mesh,
        out_type=jax.ShapeDtypeStruct((4 * B, H), jnp.float32),
        scratch_types=[
            pltpu.VMEM((ROWS_PER_W,), jnp.int32),
            pltpu.VMEM((ROWS_PER_W,), jnp.int32),
            pltpu.VMEM((CH, 8, H), jnp.float32),
            pltpu.VMEM((CH, 8, H), jnp.float32),
            pltpu.VMEM((CH, H), jnp.float32),
            pltpu.VMEM((CH, H), jnp.float32),
            pltpu.SemaphoreType.DMA,
            pltpu.SemaphoreType.DMA,
            pltpu.SemaphoreType.DMA,
            pltpu.SemaphoreType.DMA,
        ],
        compiler_params=pltpu.CompilerParams(needs_layout_passes=False),
    )(idx >> 3, idx & 7, table3)


KU = 8


def _deintl_matrix():
    ii = lax.broadcasted_iota(jnp.int32, (H, H), 0)
    jj = lax.broadcasted_iota(jnp.int32, (H, H), 1)
    src = 2 * (jj % (H // 2)) + jnp.where(jj < H // 2, 0, 1)
    return (ii == src).astype(jnp.float32)


def _score_body(rp_ref, rn_ref, ph_ref, pt_ref, nh_ref, nt_ref,
                relp_ref, out_ref, hde_ref, diff_ref):
    P = _deintl_matrix()
    hde_ref[pl.ds(0, B), :] = jnp.dot(
        ph_ref[...], P, preferred_element_type=jnp.float32)
    hde_ref[pl.ds(B, B), :] = jnp.dot(
        nh_ref[...], P, preferred_element_type=jnp.float32)

    def body(i, acc):
        b0 = i * KU
        for u in range(KU):
            b = b0 + u
            rp = rp_ref[b]
            rn = rn_ref[b]
            Rp = relp_ref[pl.ds(rp * 32, 32), :]
            Rn = relp_ref[pl.ds(rn * 32, 32), :]
            hp = hde_ref[pl.ds(b, 1), :]
            hn = hde_ref[pl.ds(B + b, 1), :]
            tp = pt_ref[pl.ds(b, 1), :]
            tn = nt_ref[pl.ds(b, 1), :]
            pe = jnp.dot(lax.slice(hp, (0, 0), (1, 32)), Rp,
                         preferred_element_type=jnp.float32)
            po = jnp.dot(lax.slice(hp, (0, 32), (1, 64)), Rp,
                         preferred_element_type=jnp.float32)
            ne = jnp.dot(lax.slice(hn, (0, 0), (1, 32)), Rn,
                         preferred_element_type=jnp.float32)
            no = jnp.dot(lax.slice(hn, (0, 32), (1, 64)), Rn,
                         preferred_element_type=jnp.float32)
            pvec = (lax.slice(pe, (0, 0), (1, H))
                    + lax.slice(po, (0, H), (1, 2 * H))) * tp
            nvec = (lax.slice(ne, (0, 0), (1, H))
                    + lax.slice(no, (0, H), (1, 2 * H))) * tn
            diff_ref[pl.ds(b, 1), :] = nvec - pvec
        return acc

    lax.fori_loop(0, B // KU, body, jnp.float32(0.0))
    d = diff_ref[...]
    s = jnp.sum(d, axis=1) + MARGIN
    out_ref[0, 0] = jnp.sum(jnp.maximum(s, 0.0))


def _score(rp, rn, ph, pt, nh, nt, rel_pair):
    return pl.pallas_call(
        _score_body,
        out_shape=jax.ShapeDtypeStruct((1, 1), jnp.float32),
        in_specs=[
            pl.BlockSpec(memory_space=pltpu.SMEM),
            pl.BlockSpec(memory_space=pltpu.SMEM),
            pl.BlockSpec(memory_space=pltpu.VMEM),
            pl.BlockSpec(memory_space=pltpu.VMEM),
            pl.BlockSpec(memory_space=pltpu.VMEM),
            pl.BlockSpec(memory_space=pltpu.VMEM),
            pl.BlockSpec(memory_space=pltpu.VMEM),
        ],
        out_specs=pl.BlockSpec(memory_space=pltpu.SMEM),
        scratch_shapes=[
            pltpu.VMEM((2 * B, H), jnp.float32),
            pltpu.VMEM((B, H), jnp.float32),
        ],
    )(rp, rn, ph, pt, nh, nt, rel_pair)


def kernel(pos_h, pos_t, pos_r, neg_h, neg_t, neg_r,
           ent_embeddings, rel_matrices):
    ph = jnp.take(ent_embeddings, pos_h, axis=0)
    pt = jnp.take(ent_embeddings, pos_t, axis=0)
    nh = jnp.take(ent_embeddings, neg_h, axis=0)
    nt = jnp.take(ent_embeddings, neg_t, axis=0)
    rel_pair = rel_matrices.reshape(REL * 32, 128)
    out = _score(pos_r.astype(jnp.int32), neg_r.astype(jnp.int32),
                 ph, pt, nh, nt, rel_pair)
    return out[0, 0]


# speedup vs baseline: 1.0026x; 1.0026x over previous
import functools

import jax
import jax.numpy as jnp
from jax import lax
from jax.experimental import pallas as pl
from jax.experimental.pallas import tpu as pltpu
from jax.experimental.pallas import tpu_sc as plsc

ENT = 1000000
REL = 1000
H = 64
B = 4096
MARGIN = 1.0

NC = 2
NS = 16
NW = NC * NS
ROWS_PER_W = 4 * B // NW


CH = 32
NCHUNK = ROWS_PER_W // CH


def _sc_gather_body(tidx_hbm, sub_hbm, table_hbm, out_hbm,
                    tidx_v, sub_v, tb0, tb1, ob0, ob1,
                    sg0, sg1, so0, so1):
    wid = lax.axis_index("s") * NC + lax.axis_index("c")
    base = wid * ROWS_PER_W
    pltpu.sync_copy(tidx_hbm.at[pl.ds(base, ROWS_PER_W)], tidx_v)
    pltpu.sync_copy(sub_hbm.at[pl.ds(base, ROWS_PER_W)], sub_v)

    tb = [tb0, tb1]
    ob = [ob0, ob1]
    sg = [sg0, sg1]
    so = [so0, so1]

    def start_gather(c):
        return pltpu.async_copy(
            table_hbm.at[tidx_v.at[pl.ds(c * CH, CH)]], tb[c % 2], sg[c % 2])

    def extract(c):
        coff = c * CH
        for half in range(CH // 16):
            jv = lax.iota(jnp.int32, 16) + (half * 16)
            sv = sub_v[pl.ds(coff + half * 16, 16)]

            def col_body(k, _):
                kv = jnp.zeros((16,), jnp.int32) + k
                v = plsc.load_gather(tb[c % 2], [jv, sv, kv])
                plsc.store_scatter(ob[c % 2], [jv, kv], v)
                return 0

            lax.fori_loop(0, H, col_body, 0)

    g = [start_gather(0)]
    w = [None, None]
    for c in range(NCHUNK):
        g[c].wait()
        if c + 1 < NCHUNK:
            g.append(start_gather(c + 1))
        if c >= 2:
            w[c % 2].wait()
        extract(c)
        w[c % 2] = pltpu.async_copy(
            ob[c % 2], out_hbm.at[pl.ds(base + c * CH, CH)], so[c % 2])
    w[0].wait()
    w[1].wait()


def _sc_gather(idx, table):
    mesh = plsc.VectorSubcoreMesh(core_axis_name="c", subcore_axis_name="s")
    table3 = table.reshape(ENT // 8, 8, H)
    return pl.kernel(
        _sc_gather_body,
        mesh=mesh,
        out_type=jax.ShapeDtypeStruct((4 * B, H), jnp.float32),
        scratch_types=[
            pltpu.VMEM((ROWS_PER_W,), jnp.int32),
            pltpu.VMEM((ROWS_PER_W,), jnp.int32),
            pltpu.VMEM((CH, 8, H), jnp.float32),
            pltpu.VMEM((CH, 8, H), jnp.float32),
            pltpu.VMEM((CH, H), jnp.float32),
            pltpu.VMEM((CH, H), jnp.float32),
            pltpu.SemaphoreType.DMA,
            pltpu.SemaphoreType.DMA,
            pltpu.SemaphoreType.DMA,
            pltpu.SemaphoreType.DMA,
        ],
        compiler_params=pltpu.CompilerParams(needs_layout_passes=False),
    )(idx >> 3, idx & 7, table3)


KU = 8


def _deintl_matrix():
    ii = lax.broadcasted_iota(jnp.int32, (H, H), 0)
    jj = lax.broadcasted_iota(jnp.int32, (H, H), 1)
    src = 2 * (jj % (H // 2)) + jnp.where(jj < H // 2, 0, 1)
    return (ii == src).astype(jnp.float32)


def _score_body(rp_ref, rn_ref, ph_ref, pt_ref, nh_ref, nt_ref,
                relp_ref, out_ref, hde_ref, diff_ref):
    P = _deintl_matrix()
    hde_ref[pl.ds(0, B), :] = jnp.dot(
        ph_ref[...], P, preferred_element_type=jnp.float32)
    hde_ref[pl.ds(B, B), :] = jnp.dot(
        nh_ref[...], P, preferred_element_type=jnp.float32)

    def body(i, acc):
        b0 = i * KU
        for u in range(KU):
            b = b0 + u
            rp = rp_ref[b]
            rn = rn_ref[b]
            Rp = relp_ref[pl.ds(rp * 32, 32), :]
            Rn = relp_ref[pl.ds(rn * 32, 32), :]
            hp = hde_ref[pl.ds(b, 1), :]
            hn = hde_ref[pl.ds(B + b, 1), :]
            tp = pt_ref[pl.ds(b, 1), :]
            tn = nt_ref[pl.ds(b, 1), :]
            pe = jnp.dot(lax.slice(hp, (0, 0), (1, 32)), Rp,
                         preferred_element_type=jnp.float32)
            po = jnp.dot(lax.slice(hp, (0, 32), (1, 64)), Rp,
                         preferred_element_type=jnp.float32)
            ne = jnp.dot(lax.slice(hn, (0, 0), (1, 32)), Rn,
                         preferred_element_type=jnp.float32)
            no = jnp.dot(lax.slice(hn, (0, 32), (1, 64)), Rn,
                         preferred_element_type=jnp.float32)
            pvec = (lax.slice(pe, (0, 0), (1, H))
                    + lax.slice(po, (0, H), (1, 2 * H))) * tp
            nvec = (lax.slice(ne, (0, 0), (1, H))
                    + lax.slice(no, (0, H), (1, 2 * H))) * tn
            diff_ref[pl.ds(b, 1), :] = nvec - pvec
        return acc

    lax.fori_loop(0, B // KU, body, jnp.float32(0.0))
    d = diff_ref[...]
    s = jnp.sum(d, axis=1) + MARGIN
    out_ref[0, 0] = jnp.sum(jnp.maximum(s, 0.0))


def _score(rp, rn, ph, pt, nh, nt, rel_pair):
    return pl.pallas_call(
        _score_body,
        out_shape=jax.ShapeDtypeStruct((1, 1), jnp.float32),
        in_specs=[
            pl.BlockSpec(memory_space=pltpu.SMEM),
            pl.BlockSpec(memory_space=pltpu.SMEM),
            pl.BlockSpec(memory_space=pltpu.VMEM),
            pl.BlockSpec(memory_space=pltpu.VMEM),
            pl.BlockSpec(memory_space=pltpu.VMEM),
            pl.BlockSpec(memory_space=pltpu.VMEM),
            pl.BlockSpec(memory_space=pltpu.VMEM),
        ],
        out_specs=pl.BlockSpec(memory_space=pltpu.SMEM),
        scratch_shapes=[
            pltpu.VMEM((2 * B, H), jnp.float32),
            pltpu.VMEM((B, H), jnp.float32),
        ],
    )(rp, rn, ph, pt, nh, nt, rel_pair)


def kernel(pos_h, pos_t, pos_r, neg_h, neg_t, neg_r,
           ent_embeddings, rel_matrices):
    ent_t = ent_embeddings.T
    ph = jnp.take(ent_t, pos_h, axis=1).T
    pt = jnp.take(ent_t, pos_t, axis=1).T
    nh = jnp.take(ent_t, neg_h, axis=1).T
    nt = jnp.take(ent_t, neg_t, axis=1).T
    rel_pair = rel_matrices.reshape(REL * 32, 128)
    out = _score(pos_r.astype(jnp.int32), neg_r.astype(jnp.int32),
                 ph, pt, nh, nt, rel_pair)
    return out[0, 0]


# speedup vs baseline: 1.0129x; 1.0103x over previous
import functools

import jax
import jax.numpy as jnp
from jax import lax
from jax.experimental import pallas as pl
from jax.experimental.pallas import tpu as pltpu
from jax.experimental.pallas import tpu_sc as plsc

ENT = 1000000
REL = 1000
H = 64
B = 4096
MARGIN = 1.0

NC = 2
NS = 16
NW = NC * NS
ROWS_PER_W = 4 * B // NW


CH = 32
NCHUNK = ROWS_PER_W // CH


def _sc_gather_body(tidx_hbm, sub_hbm, table_hbm, out_hbm,
                    tidx_v, sub_v, tb0, tb1, ob0, ob1,
                    sg0, sg1, so0, so1):
    wid = lax.axis_index("s") * NC + lax.axis_index("c")
    base = wid * ROWS_PER_W
    pltpu.sync_copy(tidx_hbm.at[pl.ds(base, ROWS_PER_W)], tidx_v)
    pltpu.sync_copy(sub_hbm.at[pl.ds(base, ROWS_PER_W)], sub_v)

    tb = [tb0, tb1]
    ob = [ob0, ob1]
    sg = [sg0, sg1]
    so = [so0, so1]

    def start_gather(c):
        return pltpu.async_copy(
            table_hbm.at[tidx_v.at[pl.ds(c * CH, CH)]], tb[c % 2], sg[c % 2])

    def extract(c):
        coff = c * CH
        for half in range(CH // 16):
            jv = lax.iota(jnp.int32, 16) + (half * 16)
            sv = sub_v[pl.ds(coff + half * 16, 16)]

            def col_body(k, _):
                kv = jnp.zeros((16,), jnp.int32) + k
                v = plsc.load_gather(tb[c % 2], [jv, sv, kv])
                plsc.store_scatter(ob[c % 2], [jv, kv], v)
                return 0

            lax.fori_loop(0, H, col_body, 0)

    g = [start_gather(0)]
    w = [None, None]
    for c in range(NCHUNK):
        g[c].wait()
        if c + 1 < NCHUNK:
            g.append(start_gather(c + 1))
        if c >= 2:
            w[c % 2].wait()
        extract(c)
        w[c % 2] = pltpu.async_copy(
            ob[c % 2], out_hbm.at[pl.ds(base + c * CH, CH)], so[c % 2])
    w[0].wait()
    w[1].wait()


def _sc_gather(idx, table):
    mesh = plsc.VectorSubcoreMesh(core_axis_name="c", subcore_axis_name="s")
    table3 = table.reshape(ENT // 8, 8, H)
    return pl.kernel(
        _sc_gather_body,
        mesh=mesh,
        out_type=jax.ShapeDtypeStruct((4 * B, H), jnp.float32),
        scratch_types=[
            pltpu.VMEM((ROWS_PER_W,), jnp.int32),
            pltpu.VMEM((ROWS_PER_W,), jnp.int32),
            pltpu.VMEM((CH, 8, H), jnp.float32),
            pltpu.VMEM((CH, 8, H), jnp.float32),
            pltpu.VMEM((CH, H), jnp.float32),
            pltpu.VMEM((CH, H), jnp.float32),
            pltpu.SemaphoreType.DMA,
            pltpu.SemaphoreType.DMA,
            pltpu.SemaphoreType.DMA,
            pltpu.SemaphoreType.DMA,
        ],
        compiler_params=pltpu.CompilerParams(needs_layout_passes=False),
    )(idx >> 3, idx & 7, table3)


KU = 8


def _deintl_matrix():
    ii = lax.broadcasted_iota(jnp.int32, (H, H), 0)
    jj = lax.broadcasted_iota(jnp.int32, (H, H), 1)
    src = 2 * (jj % (H // 2)) + jnp.where(jj < H // 2, 0, 1)
    return (ii == src).astype(jnp.float32)


def _score_body(rp_ref, rn_ref, ph_ref, pt_ref, nh_ref, nt_ref,
                relp_ref, out_ref, hde_ref, trow_ref, diff_ref):
    P = _deintl_matrix()
    ii = lax.broadcasted_iota(jnp.int32, (H, H), 0)
    jj = lax.broadcasted_iota(jnp.int32, (H, H), 1)
    I = (ii == jj).astype(jnp.float32)
    dn = (((0,), (0,)), ((), ()))
    hde_ref[pl.ds(0, B), :] = lax.dot_general(
        ph_ref[...], P, dn, preferred_element_type=jnp.float32)
    hde_ref[pl.ds(B, B), :] = lax.dot_general(
        nh_ref[...], P, dn, preferred_element_type=jnp.float32)
    trow_ref[pl.ds(0, B), :] = lax.dot_general(
        pt_ref[...], I, dn, preferred_element_type=jnp.float32)
    trow_ref[pl.ds(B, B), :] = lax.dot_general(
        nt_ref[...], I, dn, preferred_element_type=jnp.float32)

    def body(i, acc):
        b0 = i * KU
        for u in range(KU):
            b = b0 + u
            rp = rp_ref[b]
            rn = rn_ref[b]
            Rp = relp_ref[pl.ds(rp * 32, 32), :]
            Rn = relp_ref[pl.ds(rn * 32, 32), :]
            hp = hde_ref[pl.ds(b, 1), :]
            hn = hde_ref[pl.ds(B + b, 1), :]
            tp = trow_ref[pl.ds(b, 1), :]
            tn = trow_ref[pl.ds(B + b, 1), :]
            pe = jnp.dot(lax.slice(hp, (0, 0), (1, 32)), Rp,
                         preferred_element_type=jnp.float32)
            po = jnp.dot(lax.slice(hp, (0, 32), (1, 64)), Rp,
                         preferred_element_type=jnp.float32)
            ne = jnp.dot(lax.slice(hn, (0, 0), (1, 32)), Rn,
                         preferred_element_type=jnp.float32)
            no = jnp.dot(lax.slice(hn, (0, 32), (1, 64)), Rn,
                         preferred_element_type=jnp.float32)
            pvec = (lax.slice(pe, (0, 0), (1, H))
                    + lax.slice(po, (0, H), (1, 2 * H))) * tp
            nvec = (lax.slice(ne, (0, 0), (1, H))
                    + lax.slice(no, (0, H), (1, 2 * H))) * tn
            diff_ref[pl.ds(b, 1), :] = nvec - pvec
        return acc

    lax.fori_loop(0, B // KU, body, jnp.float32(0.0))
    d = diff_ref[...]
    s = jnp.sum(d, axis=1) + MARGIN
    out_ref[0, 0] = jnp.sum(jnp.maximum(s, 0.0))


def _score(rp, rn, ph, pt, nh, nt, rel_pair):
    return pl.pallas_call(
        _score_body,
        out_shape=jax.ShapeDtypeStruct((1, 1), jnp.float32),
        in_specs=[
            pl.BlockSpec(memory_space=pltpu.SMEM),
            pl.BlockSpec(memory_space=pltpu.SMEM),
            pl.BlockSpec(memory_space=pltpu.VMEM),
            pl.BlockSpec(memory_space=pltpu.VMEM),
            pl.BlockSpec(memory_space=pltpu.VMEM),
            pl.BlockSpec(memory_space=pltpu.VMEM),
            pl.BlockSpec(memory_space=pltpu.VMEM),
        ],
        out_specs=pl.BlockSpec(memory_space=pltpu.SMEM),
        scratch_shapes=[
            pltpu.VMEM((2 * B, H), jnp.float32),
            pltpu.VMEM((2 * B, H), jnp.float32),
            pltpu.VMEM((B, H), jnp.float32),
        ],
    )(rp, rn, ph, pt, nh, nt, rel_pair)


def kernel(pos_h, pos_t, pos_r, neg_h, neg_t, neg_r,
           ent_embeddings, rel_matrices):
    ent_t = ent_embeddings.T
    ph = jnp.take(ent_t, pos_h, axis=1)
    pt = jnp.take(ent_t, pos_t, axis=1)
    nh = jnp.take(ent_t, neg_h, axis=1)
    nt = jnp.take(ent_t, neg_t, axis=1)
    rel_pair = rel_matrices.reshape(REL * 32, 128)
    out = _score(pos_r.astype(jnp.int32), neg_r.astype(jnp.int32),
                 ph, pt, nh, nt, rel_pair)
    return out[0, 0]
